# baseline (speedup 1.0000x reference)
import jax
import jax.numpy as jnp
from jax.experimental import pallas as pl
from jax.experimental.pallas import tpu as pltpu

BLOCK_M = 1024
N_TOKENS = 16384
NSTEPS = N_TOKENS // BLOCK_M


def _router_kernel(x_ref, w_ref, b_ref, o_hbm, acc, sem):
    i = pl.program_id(0)
    w = w_ref[...].astype(jnp.bfloat16)
    logits = jax.lax.dot_general(
        x_ref[...].astype(jnp.bfloat16), w,
        dimension_numbers=(((1,), (1,)), ((), ())),
        preferred_element_type=jnp.float32)
    e = jnp.exp(logits + b_ref[...])
    acc[pl.ds(i * BLOCK_M, BLOCK_M), :] = (
        e * pl.reciprocal(jnp.sum(e, axis=-1, keepdims=True)))

    @pl.when(i == NSTEPS - 1)
    def _():
        out_dma = pltpu.make_async_copy(acc, o_hbm, sem)
        out_dma.start()
        out_dma.wait()


def kernel(x, W, b):
    n_tokens, embed_dim = x.shape
    n_experts = W.shape[0]
    b2 = b.reshape(1, n_experts)
    return pl.pallas_call(
        _router_kernel,
        grid=(NSTEPS,),
        in_specs=[
            pl.BlockSpec((BLOCK_M, embed_dim), lambda i: (i, 0)),
            pl.BlockSpec((n_experts, embed_dim), lambda i: (0, 0)),
            pl.BlockSpec((1, n_experts), lambda i: (0, 0)),
        ],
        out_specs=pl.BlockSpec(memory_space=pl.ANY),
        out_shape=jax.ShapeDtypeStruct((n_tokens, n_experts), jnp.float32),
        scratch_shapes=[
            pltpu.VMEM((n_tokens, n_experts), jnp.float32),
            pltpu.SemaphoreType.DMA,
        ],
        compiler_params=pltpu.CompilerParams(
            dimension_semantics=("arbitrary",),
        ),
    )(x, W, b2)
